# Initial kernel scaffold; baseline (speedup 1.0000x reference)
import jax
import jax.numpy as jnp
from jax.experimental import pallas as pl


def kernel(query, key_param, prompts):
    raise NotImplementedError("write your pallas kernel here")


# speedup vs baseline: 1.3805x; 1.3805x over previous
import functools

import jax
import jax.numpy as jnp
from jax import lax
from jax.experimental import pallas as pl
from jax.experimental.pallas import tpu as pltpu
from jax.experimental.pallas import tpu_sc as plsc

POOL = 100
SEL = 10
PLEN = 5
DIM = 768
B = 1024
EPS = 1e-8

ROW_BLOCK = 256
LANES = 128
BIG = 1e30
D2 = PLEN * DIM
BSEL = B * SEL


def _topk_body(q_ref, kt_ref, idx_ref, sum_ref):
    q = q_ref[...]
    kt = kt_ref[...]
    dot = jnp.dot(q, kt, preferred_element_type=jnp.float32)
    qn = jnp.sqrt(jnp.sum(q * q, axis=1, keepdims=True))
    kn = jnp.sqrt(jnp.sum(kt * kt, axis=0, keepdims=True))
    denom = jnp.maximum(qn * kn, EPS)
    cdist = 1.0 - dot / denom
    lane = lax.broadcasted_iota(jnp.int32, (ROW_BLOCK, LANES), 1)
    vals = jnp.where(lane < POOL, cdist, BIG)

    total = jnp.zeros((ROW_BLOCK, 1), jnp.float32)
    idx_cols = []
    for _ in range(SEL):
        m = jnp.min(vals, axis=1, keepdims=True)
        amin = jnp.min(jnp.where(vals == m, lane, LANES), axis=1, keepdims=True)
        idx_cols.append(amin)
        total = total + m
        vals = jnp.where(lane == amin, BIG, vals)
    idx_ref[...] = jnp.concatenate(idx_cols, axis=1)
    sum_ref[...] = total


def _topk_call(query, key_t):
    grid = (B // ROW_BLOCK,)
    return pl.pallas_call(
        _topk_body,
        grid=grid,
        in_specs=[
            pl.BlockSpec((ROW_BLOCK, DIM), lambda i: (i, 0)),
            pl.BlockSpec((DIM, LANES), lambda i: (0, 0)),
        ],
        out_specs=[
            pl.BlockSpec((ROW_BLOCK, SEL), lambda i: (i, 0)),
            pl.BlockSpec((ROW_BLOCK, 1), lambda i: (i, 0)),
        ],
        out_shape=[
            jax.ShapeDtypeStruct((B, SEL), jnp.int32),
            jax.ShapeDtypeStruct((B, 1), jnp.float32),
        ],
    )(query, key_t)


_NC = 2
_NS = 16
_NW = _NC * _NS
_B_PER_W = BSEL // _NW
_CHUNK = 8
_NCHUNK = _B_PER_W // _CHUNK


def _gather_call(table, idx):
    mesh = plsc.VectorSubcoreMesh(core_axis_name="c", subcore_axis_name="s")

    @functools.partial(
        pl.kernel,
        mesh=mesh,
        out_type=jax.ShapeDtypeStruct((BSEL, D2), jnp.float32),
        scratch_types=[
            pltpu.VMEM((_B_PER_W,), jnp.int32),
            pltpu.VMEM((_CHUNK, D2), jnp.float32),
            pltpu.SemaphoreType.DMA,
        ],
    )
    def k(table_hbm, idx_hbm, out_hbm, idx_v, rows_v, sem):
        wid = lax.axis_index("s") * _NC + lax.axis_index("c")
        base = pl.multiple_of(wid * _B_PER_W, 8)
        pltpu.sync_copy(idx_hbm.at[pl.ds(base, _B_PER_W)], idx_v)

        def body(c, carry):
            off = pl.multiple_of(c * _CHUNK, 8)
            pltpu.async_copy(
                table_hbm.at[idx_v.at[pl.ds(off, _CHUNK)]], rows_v, sem
            ).wait()
            pltpu.sync_copy(
                rows_v, out_hbm.at[pl.ds(base + off, _CHUNK)]
            )
            return carry

        lax.fori_loop(0, _NCHUNK, body, 0)

    return k(table, idx)


def kernel(query, key_param, prompts):
    key_t = jnp.zeros((DIM, LANES), jnp.float32).at[:, :POOL].set(key_param.T)
    topk_idx, row_sums = _topk_call(query, key_t)
    table = prompts.reshape(POOL, D2)
    flat = _gather_call(table, topk_idx.reshape(-1))
    selection = flat.reshape(B, SEL, PLEN, DIM)
    mean = jnp.sum(row_sums) / (B * SEL)
    return (mean, selection)


# speedup vs baseline: 1.6513x; 1.1961x over previous
import functools

import jax
import jax.numpy as jnp
from jax import lax
from jax.experimental import pallas as pl
from jax.experimental.pallas import tpu as pltpu
from jax.experimental.pallas import tpu_sc as plsc

POOL = 100
SEL = 10
PLEN = 5
DIM = 768
B = 1024
EPS = 1e-8

ROW_BLOCK = 256
LANES = 128
BIG = 1e30
D2 = PLEN * DIM
BSEL = B * SEL


def _topk_body(q_ref, kt_ref, idx_ref, sum_ref):
    q = q_ref[...]
    kt = kt_ref[...]
    dot = jnp.dot(q, kt, preferred_element_type=jnp.float32)
    qn = jnp.sqrt(jnp.sum(q * q, axis=1, keepdims=True))
    kn = jnp.sqrt(jnp.sum(kt * kt, axis=0, keepdims=True))
    denom = jnp.maximum(qn * kn, EPS)
    cdist = 1.0 - dot / denom
    lane = lax.broadcasted_iota(jnp.int32, (ROW_BLOCK, LANES), 1)
    vals = jnp.where(lane < POOL, cdist, BIG)

    total = jnp.zeros((ROW_BLOCK, 1), jnp.float32)
    idx_cols = []
    for _ in range(SEL):
        m = jnp.min(vals, axis=1, keepdims=True)
        amin = jnp.min(jnp.where(vals == m, lane, LANES), axis=1, keepdims=True)
        idx_cols.append(amin)
        total = total + m
        vals = jnp.where(lane == amin, BIG, vals)
    idx_ref[...] = jnp.concatenate(idx_cols, axis=1)
    sum_ref[...] = total


def _topk_call(query, key_t):
    grid = (B // ROW_BLOCK,)
    return pl.pallas_call(
        _topk_body,
        grid=grid,
        in_specs=[
            pl.BlockSpec((ROW_BLOCK, DIM), lambda i: (i, 0)),
            pl.BlockSpec((DIM, LANES), lambda i: (0, 0)),
        ],
        out_specs=[
            pl.BlockSpec((ROW_BLOCK, SEL), lambda i: (i, 0)),
            pl.BlockSpec((ROW_BLOCK, 1), lambda i: (i, 0)),
        ],
        out_shape=[
            jax.ShapeDtypeStruct((B, SEL), jnp.int32),
            jax.ShapeDtypeStruct((B, 1), jnp.float32),
        ],
    )(query, key_t)


_NC = 2
_NS = 16
_NW = _NC * _NS
_B_PER_W = BSEL // _NW
_IDXV = 16
_G = 8
_NCHUNK = _B_PER_W // _G
_NIDXV = _B_PER_W // _IDXV


def _gather_call(table, idx3):
    mesh = plsc.VectorSubcoreMesh(core_axis_name="c", subcore_axis_name="s")
    piece = 8 * D2

    @functools.partial(
        pl.kernel,
        mesh=mesh,
        out_type=jax.ShapeDtypeStruct((BSEL * D2,), jnp.float32),
        scratch_types=[
            pltpu.VMEM_SHARED((104 * D2,), jnp.float32),
            pltpu.VMEM((_NIDXV, _IDXV), jnp.int32),
            pltpu.VMEM((2 * _G * D2,), jnp.float32),
            pltpu.SemaphoreType.DMA,
            pltpu.SemaphoreType.DMA,
        ],
    )
    def k(table_hbm, idx_hbm, out_hbm, table_sh, idx_v, bufs, sem_g, sem_s):
        cid = lax.axis_index("c")
        sid = lax.axis_index("s")
        wid = sid * _NC + cid
        base = wid * _B_PER_W * D2

        for p in range(13):
            @pl.when(sid == p)
            def _stage(p=p):
                pltpu.sync_copy(
                    table_hbm.at[pl.ds(p * piece, piece)],
                    bufs.at[pl.ds(0, piece)],
                )
                pltpu.sync_copy(
                    bufs.at[pl.ds(0, piece)],
                    table_sh.at[pl.ds(p * piece, piece)],
                )

        pltpu.sync_copy(idx_hbm.at[wid], idx_v)
        plsc.subcore_barrier()

        def fire_gather(g, v, half):
            for j in range(_G):
                r = v[half * _G + j]
                pltpu.make_async_copy(
                    table_sh.at[pl.ds(r * D2, D2)],
                    bufs.at[pl.ds((g * _G + j) * D2, D2)],
                    sem_g,
                ).start()

        def drain_gather():
            for _ in range(_G):
                pltpu.make_async_copy(
                    table_sh.at[pl.ds(0, D2)], bufs.at[pl.ds(0, D2)], sem_g
                ).wait()

        def fire_scatter(cc, g):
            pltpu.make_async_copy(
                bufs.at[pl.ds(g * piece, piece)],
                out_hbm.at[pl.ds(base + cc * piece, piece)],
                sem_s,
            ).start()

        def drain_scatter():
            pltpu.make_async_copy(
                bufs.at[pl.ds(0, piece)], out_hbm.at[pl.ds(base, piece)],
                sem_s,
            ).wait()

        v0 = idx_v[0]
        fire_gather(0, v0, 0)
        drain_gather()
        fire_scatter(0, 0)
        fire_gather(1, v0, 1)

        def body(p, carry):
            v = idx_v[p + 1]
            for half, g in ((0, 0), (1, 1)):
                c = p * 2 + 2 + half
                drain_scatter()
                drain_gather()
                fire_scatter(c - 1, 1 - g)
                fire_gather(g, v, half)
            return carry

        lax.fori_loop(0, (_NCHUNK - 2) // 2, body, 0)

        drain_scatter()
        drain_gather()
        fire_scatter(_NCHUNK - 1, 1)
        drain_scatter()

    return k(table, idx3)


def kernel(query, key_param, prompts):
    key_t = jnp.zeros((DIM, LANES), jnp.float32).at[:, :POOL].set(key_param.T)
    topk_idx, row_sums = _topk_call(query, key_t)
    table = jnp.zeros((104 * D2,), jnp.float32).at[:POOL * D2].set(
        prompts.reshape(-1))
    flat = _gather_call(table, topk_idx.reshape(_NW, _NIDXV, _IDXV))
    selection = flat.reshape(B, SEL, PLEN, DIM)
    mean = jnp.sum(row_sums) / (B * SEL)
    return (mean, selection)


# speedup vs baseline: 1.6565x; 1.0032x over previous
import functools

import jax
import jax.numpy as jnp
from jax import lax
from jax.experimental import pallas as pl
from jax.experimental.pallas import tpu as pltpu
from jax.experimental.pallas import tpu_sc as plsc

POOL = 100
SEL = 10
PLEN = 5
DIM = 768
B = 1024
EPS = 1e-8

ROW_BLOCK = 256
LANES = 128
BIG = 1e30
D2 = PLEN * DIM
BSEL = B * SEL


def _topk_body(q_ref, kt_ref, idx_ref, sum_ref):
    q = q_ref[...]
    kt = kt_ref[...]
    dot = jnp.dot(q, kt, preferred_element_type=jnp.float32)
    qn = jnp.sqrt(jnp.sum(q * q, axis=1, keepdims=True))
    kn = jnp.sqrt(jnp.sum(kt * kt, axis=0, keepdims=True))
    denom = jnp.maximum(qn * kn, EPS)
    cdist = 1.0 - dot / denom
    lane = lax.broadcasted_iota(jnp.int32, (ROW_BLOCK, LANES), 1)
    vals = jnp.where(lane < POOL, cdist, BIG)

    total = jnp.zeros((ROW_BLOCK, 1), jnp.float32)
    idx_cols = []
    for _ in range(SEL):
        m = jnp.min(vals, axis=1, keepdims=True)
        amin = jnp.min(jnp.where(vals == m, lane, LANES), axis=1, keepdims=True)
        idx_cols.append(amin)
        total = total + m
        vals = jnp.where(lane == amin, BIG, vals)
    idx_ref[...] = jnp.concatenate(idx_cols, axis=1)
    sum_ref[...] = total


def _topk_call(query, key_t):
    grid = (B // ROW_BLOCK,)
    return pl.pallas_call(
        _topk_body,
        grid=grid,
        in_specs=[
            pl.BlockSpec((ROW_BLOCK, DIM), lambda i: (i, 0)),
            pl.BlockSpec((DIM, LANES), lambda i: (0, 0)),
        ],
        out_specs=[
            pl.BlockSpec((ROW_BLOCK, SEL), lambda i: (i, 0)),
            pl.BlockSpec((ROW_BLOCK, 1), lambda i: (i, 0)),
        ],
        out_shape=[
            jax.ShapeDtypeStruct((B, SEL), jnp.int32),
            jax.ShapeDtypeStruct((B, 1), jnp.float32),
        ],
    )(query, key_t)


_NC = 2
_NS = 16
_NW = _NC * _NS
_B_PER_W = BSEL // _NW
_IDXV = 16
_G = 8
_NCHUNK = _B_PER_W // _G
_NIDXV = _B_PER_W // _IDXV


def _gather_call(table, idx3):
    mesh = plsc.VectorSubcoreMesh(core_axis_name="c", subcore_axis_name="s")
    piece = 8 * D2

    @functools.partial(
        pl.kernel,
        mesh=mesh,
        out_type=jax.ShapeDtypeStruct((BSEL * D2,), jnp.float32),
        scratch_types=[
            pltpu.VMEM_SHARED((POOL * D2,), jnp.float32),
            pltpu.VMEM((_NIDXV, _IDXV), jnp.int32),
            pltpu.VMEM((2 * _G * D2,), jnp.float32),
            pltpu.SemaphoreType.DMA,
            pltpu.SemaphoreType.DMA,
        ],
    )
    def k(table_hbm, idx_hbm, out_hbm, table_sh, idx_v, bufs, sem_g, sem_s):
        cid = lax.axis_index("c")
        sid = lax.axis_index("s")
        wid = sid * _NC + cid
        base = wid * _B_PER_W * D2

        for p in range(13):
            start = (p * 8 if p < 12 else POOL - 8) * D2

            @pl.when(sid == p)
            def _stage(start=start):
                pltpu.sync_copy(
                    table_hbm.at[pl.ds(start, piece)],
                    bufs.at[pl.ds(0, piece)],
                )
                pltpu.sync_copy(
                    bufs.at[pl.ds(0, piece)],
                    table_sh.at[pl.ds(start, piece)],
                )

        pltpu.sync_copy(idx_hbm.at[wid], idx_v)
        plsc.subcore_barrier()

        def fire_gather(g, v, half):
            for j in range(_G):
                r = v[half * _G + j]
                pltpu.make_async_copy(
                    table_sh.at[pl.ds(r * D2, D2)],
                    bufs.at[pl.ds((g * _G + j) * D2, D2)],
                    sem_g,
                ).start()

        def drain_gather():
            for _ in range(_G):
                pltpu.make_async_copy(
                    table_sh.at[pl.ds(0, D2)], bufs.at[pl.ds(0, D2)], sem_g
                ).wait()

        def fire_scatter(cc, g):
            pltpu.make_async_copy(
                bufs.at[pl.ds(g * piece, piece)],
                out_hbm.at[pl.ds(base + cc * piece, piece)],
                sem_s,
            ).start()

        def drain_scatter():
            pltpu.make_async_copy(
                bufs.at[pl.ds(0, piece)], out_hbm.at[pl.ds(base, piece)],
                sem_s,
            ).wait()

        v0 = idx_v[0]
        fire_gather(0, v0, 0)
        drain_gather()
        fire_scatter(0, 0)
        fire_gather(1, v0, 1)

        def body(p, carry):
            v = idx_v[p + 1]
            for half, g in ((0, 0), (1, 1)):
                c = p * 2 + 2 + half
                drain_scatter()
                drain_gather()
                fire_scatter(c - 1, 1 - g)
                fire_gather(g, v, half)
            return carry

        lax.fori_loop(0, (_NCHUNK - 2) // 2, body, 0)

        drain_scatter()
        drain_gather()
        fire_scatter(_NCHUNK - 1, 1)
        drain_scatter()

    return k(table, idx3)


def kernel(query, key_param, prompts):
    key_t = jnp.zeros((DIM, LANES), jnp.float32).at[:, :POOL].set(key_param.T)
    topk_idx, row_sums = _topk_call(query, key_t)
    table = prompts.reshape(-1)
    flat = _gather_call(table, topk_idx.reshape(_NW, _NIDXV, _IDXV))
    selection = flat.reshape(B, SEL, PLEN, DIM)
    mean = jnp.sum(row_sums) / (B * SEL)
    return (mean, selection)
